# baseline (speedup 1.0000x reference)
import functools

import jax
import jax.numpy as jnp
from jax import lax
from jax.experimental import pallas as pl
from jax.experimental.pallas import tpu as pltpu
from jax.experimental.pallas import tpu_sc as plsc

BT = 64
NC, NS = 2, 16
NW = NC * NS


def _routing_body(x_ref, gw_ref, pos_ref, te_ref, lv_ref):
    T = x_ref.shape[0]
    E = gw_ref.shape[0]
    NT = te_ref.shape[0]
    x = x_ref[...]
    gw = gw_ref[...]
    logits = lax.dot_general(
        x.astype(jnp.bfloat16), gw.astype(jnp.bfloat16),
        (((1,), (1,)), ((), ())),
        preferred_element_type=jnp.float32,
    )
    m = jnp.max(logits, axis=1, keepdims=True)
    lane_e = lax.broadcasted_iota(jnp.int32, (T, E), 1)
    eid = jnp.min(jnp.where(logits == m, lane_e, E), axis=1, keepdims=True)

    onehot = (eid == lane_e).astype(jnp.float32)
    counts = jnp.sum(onehot, axis=0, keepdims=True)
    tiles = jnp.floor((counts + (BT - 1)) * (1.0 / BT))
    padded = tiles * BT
    r_e = lax.broadcasted_iota(jnp.int32, (E, E), 0)
    c_e = lax.broadcasted_iota(jnp.int32, (E, E), 1)
    tri = (r_e < c_e).astype(jnp.float32)
    offs = lax.dot_general(
        padded, tri, (((1,), (0,)), ((), ())),
        preferred_element_type=jnp.float32,
        precision=lax.Precision.HIGHEST,
    )

    C = 16
    CH = T // C
    oh3 = onehot.reshape(C, CH, E)
    r_t = lax.broadcasted_iota(jnp.int32, (CH, CH), 0)
    c_t = lax.broadcasted_iota(jnp.int32, (CH, CH), 1)
    tri_t = jnp.broadcast_to(
        (c_t < r_t).astype(jnp.float32)[None], (C, CH, CH))
    rank_in = lax.dot_general(
        tri_t, oh3, (((2,), (1,)), ((0,), (0,))),
        preferred_element_type=jnp.float32)
    cc = jnp.sum(oh3, axis=1)
    r_c = lax.broadcasted_iota(jnp.int32, (C, C), 0)
    c_c = lax.broadcasted_iota(jnp.int32, (C, C), 1)
    tri_c = (c_c < r_c).astype(jnp.float32)
    base_c = lax.dot_general(
        tri_c, cc, (((1,), (0,)), ((), ())),
        preferred_element_type=jnp.float32)
    pos3 = rank_in + base_c[:, None, :] + offs.reshape(1, 1, E)
    pos_ref[...] = jnp.sum(pos3 * oh3, axis=2).astype(jnp.int32)

    starts = (lax.broadcasted_iota(jnp.int32, (NT, 1), 0) * BT).astype(jnp.float32)
    cmp = (offs <= starts).astype(jnp.int32)
    te_ref[...] = jnp.sum(cmp, axis=1, keepdims=True) - 1
    total = jnp.sum(padded, axis=1, keepdims=True)
    lv_ref[...] = (starts < total).astype(jnp.int32)


def _routing(x, gate_w, NT):
    T, _ = x.shape
    pos, te, lv = pl.pallas_call(
        _routing_body,
        out_shape=(
            jax.ShapeDtypeStruct((16, T // 16), jnp.int32),
            jax.ShapeDtypeStruct((NT, 1), jnp.int32),
            jax.ShapeDtypeStruct((NT, 1), jnp.int32),
        ),
    )(x, gate_w)
    return pos.reshape(T), te.reshape(NT), lv.reshape(NT)


def _dispatch_body(x_hbm, pos_hbm, xp_hbm, idx_v, rows_v, sem):
    bpw = idx_v.shape[0]
    wid = lax.axis_index("s") * NC + lax.axis_index("c")
    base = wid * bpw
    pltpu.sync_copy(pos_hbm.at[pl.ds(base, bpw)], idx_v)
    pltpu.sync_copy(x_hbm.at[pl.ds(base, bpw)], rows_v)
    pltpu.async_copy(rows_v, xp_hbm.at[idx_v], sem).wait()


def _dispatch(x, pos, NPAD):
    T, D = x.shape
    bpw = T // NW
    mesh = plsc.VectorSubcoreMesh(core_axis_name="c", subcore_axis_name="s")
    k = functools.partial(
        pl.kernel,
        mesh=mesh,
        out_type=jax.ShapeDtypeStruct((NPAD, D), jnp.float32),
        scratch_types=[
            pltpu.VMEM((bpw,), jnp.int32),
            pltpu.VMEM((bpw, D), jnp.float32),
            pltpu.SemaphoreType.DMA,
        ],
    )(_dispatch_body)
    return k(x, pos)


def _ffn_body(te_ref, lv_ref, x_ref, w1_ref, w3_ref, w2_ref, o_ref):
    del te_ref
    i = pl.program_id(0)

    @pl.when(lv_ref[i] != 0)
    def _():
        x = x_ref[...]
        H = w1_ref.shape[1]
        H2 = H // 2

        def dots(lo, hi):
            h1 = lax.dot_general(x, w1_ref[0, lo:hi], (((1,), (1,)), ((), ())),
                                 preferred_element_type=jnp.float32,
                                 precision=lax.Precision.DEFAULT)
            h3 = lax.dot_general(x, w3_ref[0, lo:hi], (((1,), (1,)), ((), ())),
                                 preferred_element_type=jnp.float32,
                                 precision=lax.Precision.DEFAULT)
            h = h1 * (1.0 / (1.0 + jnp.exp(-h1))) * h3
            return lax.dot_general(h, w2_ref[0, :, lo:hi],
                                   (((1,), (1,)), ((), ())),
                                   preferred_element_type=jnp.float32,
                                   precision=lax.Precision.DEFAULT)

        o_ref[...] = dots(0, H2) + dots(H2, H)


def _ffn(te, lv, x_pad, W1, W2, W3):
    NPAD, D = x_pad.shape
    E, H, _ = W1.shape
    NT = NPAD // BT
    grid_spec = pltpu.PrefetchScalarGridSpec(
        num_scalar_prefetch=2,
        grid=(NT,),
        in_specs=[
            pl.BlockSpec((BT, D), lambda i, te_r, lv_r: (i * lv_r[i], 0)),
            pl.BlockSpec((1, H, D), lambda i, te_r, lv_r: (te_r[i], 0, 0)),
            pl.BlockSpec((1, H, D), lambda i, te_r, lv_r: (te_r[i], 0, 0)),
            pl.BlockSpec((1, D, H), lambda i, te_r, lv_r: (te_r[i], 0, 0)),
        ],
        out_specs=pl.BlockSpec(
            (BT, D), lambda i, te_r, lv_r: (jnp.where(lv_r[i] != 0, i, NT), 0)),
    )
    return pl.pallas_call(
        _ffn_body,
        grid_spec=grid_spec,
        out_shape=jax.ShapeDtypeStruct((NPAD + BT, D), jnp.float32),
    )(te, lv, x_pad, W1, W3, W2)


def _combine_body(op_hbm, pos_hbm, out_hbm, idx_v, rows_v, sem):
    bpw = idx_v.shape[0]
    wid = lax.axis_index("s") * NC + lax.axis_index("c")
    base = wid * bpw
    pltpu.sync_copy(pos_hbm.at[pl.ds(base, bpw)], idx_v)
    pltpu.async_copy(op_hbm.at[idx_v], rows_v, sem).wait()
    pltpu.sync_copy(rows_v, out_hbm.at[pl.ds(base, bpw)])


def _combine(out_pad, pos, T):
    _, D = out_pad.shape
    bpw = T // NW
    mesh = plsc.VectorSubcoreMesh(core_axis_name="c", subcore_axis_name="s")
    k = functools.partial(
        pl.kernel,
        mesh=mesh,
        out_type=jax.ShapeDtypeStruct((T, D), jnp.float32),
        scratch_types=[
            pltpu.VMEM((bpw,), jnp.int32),
            pltpu.VMEM((bpw, D), jnp.float32),
            pltpu.SemaphoreType.DMA,
        ],
    )(_combine_body)
    return k(out_pad, pos)


def kernel(inputs, gate_w, W1, W2, W3):
    B, S, D = inputs.shape
    T = B * S
    E = gate_w.shape[0]
    NPAD = T + E * BT
    NT = NPAD // BT
    x = inputs.reshape(T, D)
    pos, te, lv = _routing(x, gate_w, NT)
    x_pad = _dispatch(x, pos, NPAD)
    out_pad = _ffn(te, lv, x_pad, W1, W2, W3)
    out = _combine(out_pad, pos, T)
    return out.reshape(B, S, D)


# speedup vs baseline: 1.0386x; 1.0386x over previous
import functools

import jax
import jax.numpy as jnp
from jax import lax
from jax.experimental import pallas as pl
from jax.experimental.pallas import tpu as pltpu
from jax.experimental.pallas import tpu_sc as plsc

BT = 64
NC, NS = 2, 16
NW = NC * NS


def _routing_body(x_ref, gw_ref, pos_ref, te_ref, lv_ref):
    T = x_ref.shape[0]
    E = gw_ref.shape[0]
    NT = te_ref.shape[0]
    x = x_ref[...]
    gw = gw_ref[...]
    logits = lax.dot_general(
        x.astype(jnp.bfloat16), gw.astype(jnp.bfloat16),
        (((1,), (1,)), ((), ())),
        preferred_element_type=jnp.float32,
    )
    m = jnp.max(logits, axis=1, keepdims=True)
    lane_e = lax.broadcasted_iota(jnp.int32, (T, E), 1)
    eid = jnp.min(jnp.where(logits == m, lane_e, E), axis=1, keepdims=True)

    onehot = (eid == lane_e).astype(jnp.float32)
    counts = jnp.sum(onehot, axis=0, keepdims=True)
    tiles = jnp.floor((counts + (BT - 1)) * (1.0 / BT))
    padded = tiles * BT
    r_e = lax.broadcasted_iota(jnp.int32, (E, E), 0)
    c_e = lax.broadcasted_iota(jnp.int32, (E, E), 1)
    tri = (r_e < c_e).astype(jnp.float32)
    offs = lax.dot_general(
        padded, tri, (((1,), (0,)), ((), ())),
        preferred_element_type=jnp.float32,
        precision=lax.Precision.HIGHEST,
    )

    C = 16
    CH = T // C
    oh3 = onehot.reshape(C, CH, E)
    r_t = lax.broadcasted_iota(jnp.int32, (CH, CH), 0)
    c_t = lax.broadcasted_iota(jnp.int32, (CH, CH), 1)
    tri_t = jnp.broadcast_to(
        (c_t < r_t).astype(jnp.float32)[None], (C, CH, CH))
    rank_in = lax.dot_general(
        tri_t, oh3, (((2,), (1,)), ((0,), (0,))),
        preferred_element_type=jnp.float32)
    cc = jnp.sum(oh3, axis=1)
    r_c = lax.broadcasted_iota(jnp.int32, (C, C), 0)
    c_c = lax.broadcasted_iota(jnp.int32, (C, C), 1)
    tri_c = (c_c < r_c).astype(jnp.float32)
    base_c = lax.dot_general(
        tri_c, cc, (((1,), (0,)), ((), ())),
        preferred_element_type=jnp.float32)
    pos3 = rank_in + base_c[:, None, :] + offs.reshape(1, 1, E)
    pos_ref[...] = jnp.sum(pos3 * oh3, axis=2).astype(jnp.int32)

    starts = (lax.broadcasted_iota(jnp.int32, (NT, 1), 0) * BT).astype(jnp.float32)
    cmp = (offs <= starts).astype(jnp.int32)
    te_ref[...] = jnp.sum(cmp, axis=1, keepdims=True) - 1
    total = jnp.sum(padded, axis=1, keepdims=True)
    lv_ref[...] = (starts < total).astype(jnp.int32)


def _routing(x, gate_w, NT):
    T, _ = x.shape
    pos, te, lv = pl.pallas_call(
        _routing_body,
        out_shape=(
            jax.ShapeDtypeStruct((16, T // 16), jnp.int32),
            jax.ShapeDtypeStruct((NT, 1), jnp.int32),
            jax.ShapeDtypeStruct((NT, 1), jnp.int32),
        ),
    )(x, gate_w)
    return pos.reshape(T), te.reshape(NT), lv.reshape(NT)


def _dispatch_body(x_hbm, pos_hbm, xp_hbm, idx_v, rows_v, sem):
    bpw = idx_v.shape[0]
    wid = lax.axis_index("s") * NC + lax.axis_index("c")
    base = wid * bpw
    pltpu.sync_copy(pos_hbm.at[pl.ds(base, bpw)], idx_v)
    pltpu.sync_copy(x_hbm.at[pl.ds(base, bpw)], rows_v)
    pltpu.async_copy(rows_v, xp_hbm.at[idx_v], sem).wait()


def _dispatch(x, pos, NPAD):
    T, D = x.shape
    bpw = T // NW
    mesh = plsc.VectorSubcoreMesh(core_axis_name="c", subcore_axis_name="s")
    k = functools.partial(
        pl.kernel,
        mesh=mesh,
        out_type=jax.ShapeDtypeStruct((NPAD, D), jnp.float32),
        scratch_types=[
            pltpu.VMEM((bpw,), jnp.int32),
            pltpu.VMEM((bpw, D), jnp.float32),
            pltpu.SemaphoreType.DMA,
        ],
    )(_dispatch_body)
    return k(x, pos)


def _ffn_body(te_ref, lv_ref, x_ref, w1_ref, w3_ref, w2_ref, o_ref):
    del te_ref
    i = pl.program_id(0)

    @pl.when(lv_ref[i] != 0)
    def _():
        x = x_ref[...]
        h1 = lax.dot_general(x, w1_ref[0], (((1,), (1,)), ((), ())),
                             preferred_element_type=jnp.float32,
                             precision=lax.Precision.DEFAULT)
        h3 = lax.dot_general(x, w3_ref[0], (((1,), (1,)), ((), ())),
                             preferred_element_type=jnp.float32,
                             precision=lax.Precision.DEFAULT)
        h = h1 * (1.0 / (1.0 + jnp.exp(-h1))) * h3
        o_ref[...] = lax.dot_general(h, w2_ref[0], (((1,), (1,)), ((), ())),
                                     preferred_element_type=jnp.float32,
                                     precision=lax.Precision.DEFAULT)


def _ffn(te, lv, x_pad, W1, W2, W3):
    NPAD, D = x_pad.shape
    E, H, _ = W1.shape
    NT = NPAD // BT
    grid_spec = pltpu.PrefetchScalarGridSpec(
        num_scalar_prefetch=2,
        grid=(NT,),
        in_specs=[
            pl.BlockSpec((BT, D), lambda i, te_r, lv_r: (i * lv_r[i], 0)),
            pl.BlockSpec((1, H, D), lambda i, te_r, lv_r: (te_r[i], 0, 0)),
            pl.BlockSpec((1, H, D), lambda i, te_r, lv_r: (te_r[i], 0, 0)),
            pl.BlockSpec((1, D, H), lambda i, te_r, lv_r: (te_r[i], 0, 0)),
        ],
        out_specs=pl.BlockSpec(
            (BT, D), lambda i, te_r, lv_r: (jnp.where(lv_r[i] != 0, i, NT), 0)),
    )
    return pl.pallas_call(
        _ffn_body,
        grid_spec=grid_spec,
        out_shape=jax.ShapeDtypeStruct((NPAD + BT, D), jnp.float32),
    )(te, lv, x_pad, W1, W3, W2)


def _combine_body(op_hbm, pos_hbm, out_hbm, idx_v, rows_v, sem):
    bpw = idx_v.shape[0]
    wid = lax.axis_index("s") * NC + lax.axis_index("c")
    base = wid * bpw
    pltpu.sync_copy(pos_hbm.at[pl.ds(base, bpw)], idx_v)
    pltpu.async_copy(op_hbm.at[idx_v], rows_v, sem).wait()
    pltpu.sync_copy(rows_v, out_hbm.at[pl.ds(base, bpw)])


def _combine(out_pad, pos, T):
    _, D = out_pad.shape
    bpw = T // NW
    mesh = plsc.VectorSubcoreMesh(core_axis_name="c", subcore_axis_name="s")
    k = functools.partial(
        pl.kernel,
        mesh=mesh,
        out_type=jax.ShapeDtypeStruct((T, D), jnp.float32),
        scratch_types=[
            pltpu.VMEM((bpw,), jnp.int32),
            pltpu.VMEM((bpw, D), jnp.float32),
            pltpu.SemaphoreType.DMA,
        ],
    )(_combine_body)
    return k(out_pad, pos)


def kernel(inputs, gate_w, W1, W2, W3):
    B, S, D = inputs.shape
    T = B * S
    E = gate_w.shape[0]
    NPAD = T + E * BT
    NT = NPAD // BT
    x = inputs.reshape(T, D)
    pos, te, lv = _routing(x, gate_w, NT)
    x_pad = _dispatch(x, pos, NPAD)
    out_pad = _ffn(te, lv, x_pad, W1, W2, W3)
    out = _combine(out_pad, pos, T)
    return out.reshape(B, S, D)


# speedup vs baseline: 1.0407x; 1.0020x over previous
import functools

import jax
import jax.numpy as jnp
from jax import lax
from jax.experimental import pallas as pl
from jax.experimental.pallas import tpu as pltpu
from jax.experimental.pallas import tpu_sc as plsc

BT = 64
NC, NS = 2, 16
NW = NC * NS


def _routing_body(x_ref, gw_ref, pos_ref, te_ref, lv_ref):
    T = x_ref.shape[0]
    E = gw_ref.shape[0]
    NT = te_ref.shape[0]
    x = x_ref[...]
    gw = gw_ref[...]
    logits = lax.dot_general(
        x, gw, (((1,), (1,)), ((), ())),
        preferred_element_type=jnp.float32,
        precision=lax.Precision.DEFAULT,
    )
    m = jnp.max(logits, axis=1, keepdims=True)
    lane_e = lax.broadcasted_iota(jnp.int32, (T, E), 1)
    eid = jnp.min(jnp.where(logits == m, lane_e, E), axis=1, keepdims=True)

    onehot = (eid == lane_e).astype(jnp.float32)
    counts = jnp.sum(onehot, axis=0, keepdims=True)
    tiles = jnp.floor((counts + (BT - 1)) * (1.0 / BT))
    padded = tiles * BT
    r_e = lax.broadcasted_iota(jnp.int32, (E, E), 0)
    c_e = lax.broadcasted_iota(jnp.int32, (E, E), 1)
    tri = (r_e < c_e).astype(jnp.float32)
    offs = lax.dot_general(
        padded, tri, (((1,), (0,)), ((), ())),
        preferred_element_type=jnp.float32,
        precision=lax.Precision.HIGHEST,
    )

    C = 16
    CH = T // C
    oh3 = onehot.reshape(C, CH, E)
    r_t = lax.broadcasted_iota(jnp.int32, (CH, CH), 0)
    c_t = lax.broadcasted_iota(jnp.int32, (CH, CH), 1)
    tri_t = jnp.broadcast_to(
        (c_t < r_t).astype(jnp.float32)[None], (C, CH, CH))
    rank_in = lax.dot_general(
        tri_t, oh3, (((2,), (1,)), ((0,), (0,))),
        preferred_element_type=jnp.float32)
    cc = jnp.sum(oh3, axis=1)
    r_c = lax.broadcasted_iota(jnp.int32, (C, C), 0)
    c_c = lax.broadcasted_iota(jnp.int32, (C, C), 1)
    tri_c = (c_c < r_c).astype(jnp.float32)
    base_c = lax.dot_general(
        tri_c, cc, (((1,), (0,)), ((), ())),
        preferred_element_type=jnp.float32)
    pos3 = rank_in + base_c[:, None, :] + offs.reshape(1, 1, E)
    pos_ref[...] = jnp.sum(pos3 * oh3, axis=2).astype(jnp.int32)

    starts = (lax.broadcasted_iota(jnp.int32, (NT, 1), 0) * BT).astype(jnp.float32)
    cmp = (offs <= starts).astype(jnp.int32)
    te_ref[...] = jnp.sum(cmp, axis=1, keepdims=True) - 1
    total = jnp.sum(padded, axis=1, keepdims=True)
    lv_ref[...] = (starts < total).astype(jnp.int32)


def _routing(x, gate_w, NT):
    T, _ = x.shape
    pos, te, lv = pl.pallas_call(
        _routing_body,
        out_shape=(
            jax.ShapeDtypeStruct((16, T // 16), jnp.int32),
            jax.ShapeDtypeStruct((NT, 1), jnp.int32),
            jax.ShapeDtypeStruct((NT, 1), jnp.int32),
        ),
    )(x, gate_w)
    return pos.reshape(T), te.reshape(NT), lv.reshape(NT)


def _dispatch_body(x_hbm, pos_hbm, xp_hbm, idx_v, rows_v, sem):
    bpw = idx_v.shape[0]
    wid = lax.axis_index("s") * NC + lax.axis_index("c")
    base = wid * bpw
    pltpu.sync_copy(pos_hbm.at[pl.ds(base, bpw)], idx_v)
    pltpu.sync_copy(x_hbm.at[pl.ds(base, bpw)], rows_v)
    pltpu.async_copy(rows_v, xp_hbm.at[idx_v], sem).wait()


def _dispatch(x, pos, NPAD):
    T, D = x.shape
    bpw = T // NW
    mesh = plsc.VectorSubcoreMesh(core_axis_name="c", subcore_axis_name="s")
    k = functools.partial(
        pl.kernel,
        mesh=mesh,
        out_type=jax.ShapeDtypeStruct((NPAD, D), jnp.float32),
        scratch_types=[
            pltpu.VMEM((bpw,), jnp.int32),
            pltpu.VMEM((bpw, D), jnp.float32),
            pltpu.SemaphoreType.DMA,
        ],
    )(_dispatch_body)
    return k(x, pos)


def _ffn_body(te_ref, lv_ref, x_ref, w1_ref, w3_ref, w2_ref, o_ref):
    del te_ref
    i = pl.program_id(0)

    @pl.when(lv_ref[i] != 0)
    def _():
        x = x_ref[...]
        h1 = lax.dot_general(x, w1_ref[0], (((1,), (1,)), ((), ())),
                             preferred_element_type=jnp.float32,
                             precision=lax.Precision.DEFAULT)
        h3 = lax.dot_general(x, w3_ref[0], (((1,), (1,)), ((), ())),
                             preferred_element_type=jnp.float32,
                             precision=lax.Precision.DEFAULT)
        h = h1 * (1.0 / (1.0 + jnp.exp(-h1))) * h3
        o_ref[...] = lax.dot_general(h, w2_ref[0], (((1,), (1,)), ((), ())),
                                     preferred_element_type=jnp.float32,
                                     precision=lax.Precision.DEFAULT)


def _ffn(te, lv, x_pad, W1, W2, W3):
    NPAD, D = x_pad.shape
    E, H, _ = W1.shape
    NT = NPAD // BT
    grid_spec = pltpu.PrefetchScalarGridSpec(
        num_scalar_prefetch=2,
        grid=(NT,),
        in_specs=[
            pl.BlockSpec((BT, D), lambda i, te_r, lv_r: (i * lv_r[i], 0)),
            pl.BlockSpec((1, H, D), lambda i, te_r, lv_r: (te_r[i], 0, 0)),
            pl.BlockSpec((1, H, D), lambda i, te_r, lv_r: (te_r[i], 0, 0)),
            pl.BlockSpec((1, D, H), lambda i, te_r, lv_r: (te_r[i], 0, 0)),
        ],
        out_specs=pl.BlockSpec(
            (BT, D), lambda i, te_r, lv_r: (jnp.where(lv_r[i] != 0, i, NT), 0)),
    )
    return pl.pallas_call(
        _ffn_body,
        grid_spec=grid_spec,
        out_shape=jax.ShapeDtypeStruct((NPAD + BT, D), jnp.float32),
    )(te, lv, x_pad, W1, W3, W2)


def _combine_body(op_hbm, pos_hbm, out_hbm, idx_v, rows_v, sem):
    bpw = idx_v.shape[0]
    wid = lax.axis_index("s") * NC + lax.axis_index("c")
    base = wid * bpw
    pltpu.sync_copy(pos_hbm.at[pl.ds(base, bpw)], idx_v)
    pltpu.async_copy(op_hbm.at[idx_v], rows_v, sem).wait()
    pltpu.sync_copy(rows_v, out_hbm.at[pl.ds(base, bpw)])


def _combine(out_pad, pos, T):
    _, D = out_pad.shape
    bpw = T // NW
    mesh = plsc.VectorSubcoreMesh(core_axis_name="c", subcore_axis_name="s")
    k = functools.partial(
        pl.kernel,
        mesh=mesh,
        out_type=jax.ShapeDtypeStruct((T, D), jnp.float32),
        scratch_types=[
            pltpu.VMEM((bpw,), jnp.int32),
            pltpu.VMEM((bpw, D), jnp.float32),
            pltpu.SemaphoreType.DMA,
        ],
    )(_combine_body)
    return k(out_pad, pos)


def kernel(inputs, gate_w, W1, W2, W3):
    B, S, D = inputs.shape
    T = B * S
    E = gate_w.shape[0]
    NPAD = T + E * BT
    NT = NPAD // BT
    x = inputs.reshape(T, D)
    pos, te, lv = _routing(x, gate_w, NT)
    x_pad = _dispatch(x, pos, NPAD)
    out_pad = _ffn(te, lv, x_pad, W1, W2, W3)
    out = _combine(out_pad, pos, T)
    return out.reshape(B, S, D)


# speedup vs baseline: 1.0435x; 1.0028x over previous
import functools

import jax
import jax.numpy as jnp
from jax import lax
from jax.experimental import pallas as pl
from jax.experimental.pallas import tpu as pltpu
from jax.experimental.pallas import tpu_sc as plsc

BT = 64
NC, NS = 2, 16
NW = NC * NS


def _routing_body(x_ref, gw_ref, pos_ref, meta_ref):
    T = x_ref.shape[0]
    E = gw_ref.shape[0]
    x = x_ref[...]
    gw = gw_ref[...]
    logits = lax.dot_general(
        x, gw, (((1,), (1,)), ((), ())),
        preferred_element_type=jnp.float32,
        precision=lax.Precision.DEFAULT,
    )
    m = jnp.max(logits, axis=1, keepdims=True)
    lane_e = lax.broadcasted_iota(jnp.int32, (T, E), 1)
    eid = jnp.min(jnp.where(logits == m, lane_e, E), axis=1, keepdims=True)

    onehot = (eid == lane_e).astype(jnp.float32)
    counts = jnp.sum(onehot, axis=0, keepdims=True)
    tiles = jnp.floor((counts + (BT - 1)) * (1.0 / BT))
    padded = tiles * BT
    r_e = lax.broadcasted_iota(jnp.int32, (E, E), 0)
    c_e = lax.broadcasted_iota(jnp.int32, (E, E), 1)
    tri = (r_e < c_e).astype(jnp.float32)
    offs = lax.dot_general(
        padded, tri, (((1,), (0,)), ((), ())),
        preferred_element_type=jnp.float32,
        precision=lax.Precision.HIGHEST,
    )

    C = 16
    CH = T // C
    oh3 = onehot.reshape(C, CH, E)
    r_t = lax.broadcasted_iota(jnp.int32, (CH, CH), 0)
    c_t = lax.broadcasted_iota(jnp.int32, (CH, CH), 1)
    tri_t = jnp.broadcast_to(
        (c_t < r_t).astype(jnp.float32)[None], (C, CH, CH))
    rank_in = lax.dot_general(
        tri_t, oh3, (((2,), (1,)), ((0,), (0,))),
        preferred_element_type=jnp.float32)
    cc = jnp.sum(oh3, axis=1)
    r_c = lax.broadcasted_iota(jnp.int32, (C, C), 0)
    c_c = lax.broadcasted_iota(jnp.int32, (C, C), 1)
    tri_c = (c_c < r_c).astype(jnp.float32)
    base_c = lax.dot_general(
        tri_c, cc, (((1,), (0,)), ((), ())),
        preferred_element_type=jnp.float32)
    pos3 = rank_in + base_c[:, None, :] + offs.reshape(1, 1, E)
    pos_ref[...] = jnp.sum(pos3 * oh3, axis=2).astype(jnp.int32)

    meta_ref[...] = jnp.concatenate(
        [jnp.broadcast_to(offs, (8, E)),
         jnp.broadcast_to(padded, (8, E))], axis=0).astype(jnp.int32)


def _telv_body(meta_ref, te_ref, lv_ref):
    NT = te_ref.shape[0]
    E = meta_ref.shape[1]
    offs = meta_ref[0:1, :].astype(jnp.float32)
    padded = meta_ref[8:9, :].astype(jnp.float32)
    starts = (lax.broadcasted_iota(jnp.int32, (NT, 1), 0) * BT).astype(jnp.float32)
    cmp = (offs <= starts).astype(jnp.int32)
    te_ref[...] = jnp.sum(cmp, axis=1, keepdims=True) - 1
    total = jnp.sum(padded, axis=1, keepdims=True)
    lv_ref[...] = (starts < total).astype(jnp.int32)


def _routing(x, gate_w, NT):
    T, _ = x.shape
    E = gate_w.shape[0]
    pos, meta = pl.pallas_call(
        _routing_body,
        out_shape=(
            jax.ShapeDtypeStruct((16, T // 16), jnp.int32),
            jax.ShapeDtypeStruct((16, E), jnp.int32),
        ),
    )(x, gate_w)
    te, lv = pl.pallas_call(
        _telv_body,
        out_shape=(
            jax.ShapeDtypeStruct((NT, 1), jnp.int32),
            jax.ShapeDtypeStruct((NT, 1), jnp.int32),
        ),
    )(meta)
    return pos.reshape(T), te.reshape(NT), lv.reshape(NT)


def _dispatch_body(x_hbm, pos_hbm, xp_hbm, idx_v, rows_v, sem):
    bpw = idx_v.shape[0]
    wid = lax.axis_index("s") * NC + lax.axis_index("c")
    base = wid * bpw
    pltpu.sync_copy(pos_hbm.at[pl.ds(base, bpw)], idx_v)
    pltpu.sync_copy(x_hbm.at[pl.ds(base, bpw)], rows_v)
    pltpu.async_copy(rows_v, xp_hbm.at[idx_v], sem).wait()


def _dispatch(x, pos, NPAD):
    T, D = x.shape
    bpw = T // NW
    mesh = plsc.VectorSubcoreMesh(core_axis_name="c", subcore_axis_name="s")
    k = functools.partial(
        pl.kernel,
        mesh=mesh,
        out_type=jax.ShapeDtypeStruct((NPAD, D), jnp.float32),
        scratch_types=[
            pltpu.VMEM((bpw,), jnp.int32),
            pltpu.VMEM((bpw, D), jnp.float32),
            pltpu.SemaphoreType.DMA,
        ],
    )(_dispatch_body)
    return k(x, pos)


def _ffn_body(te_ref, lv_ref, x_ref, w1_ref, w3_ref, w2_ref, o_ref):
    del te_ref
    i = pl.program_id(0)

    @pl.when(lv_ref[i] != 0)
    def _():
        x = x_ref[...]
        h1 = lax.dot_general(x, w1_ref[0], (((1,), (1,)), ((), ())),
                             preferred_element_type=jnp.float32,
                             precision=lax.Precision.DEFAULT)
        h3 = lax.dot_general(x, w3_ref[0], (((1,), (1,)), ((), ())),
                             preferred_element_type=jnp.float32,
                             precision=lax.Precision.DEFAULT)
        h = h1 * (1.0 / (1.0 + jnp.exp(-h1))) * h3
        o_ref[...] = lax.dot_general(h, w2_ref[0], (((1,), (1,)), ((), ())),
                                     preferred_element_type=jnp.float32,
                                     precision=lax.Precision.DEFAULT)


def _ffn(te, lv, x_pad, W1, W2, W3):
    NPAD, D = x_pad.shape
    E, H, _ = W1.shape
    NT = NPAD // BT
    grid_spec = pltpu.PrefetchScalarGridSpec(
        num_scalar_prefetch=2,
        grid=(NT,),
        in_specs=[
            pl.BlockSpec((BT, D), lambda i, te_r, lv_r: (i * lv_r[i], 0)),
            pl.BlockSpec((1, H, D), lambda i, te_r, lv_r: (te_r[i], 0, 0)),
            pl.BlockSpec((1, H, D), lambda i, te_r, lv_r: (te_r[i], 0, 0)),
            pl.BlockSpec((1, D, H), lambda i, te_r, lv_r: (te_r[i], 0, 0)),
        ],
        out_specs=pl.BlockSpec(
            (BT, D), lambda i, te_r, lv_r: (jnp.where(lv_r[i] != 0, i, NT), 0)),
    )
    return pl.pallas_call(
        _ffn_body,
        grid_spec=grid_spec,
        out_shape=jax.ShapeDtypeStruct((NPAD + BT, D), jnp.float32),
    )(te, lv, x_pad, W1, W3, W2)


def _combine_body(op_hbm, pos_hbm, out_hbm, idx_v, rows_v, sem):
    bpw = idx_v.shape[0]
    wid = lax.axis_index("s") * NC + lax.axis_index("c")
    base = wid * bpw
    pltpu.sync_copy(pos_hbm.at[pl.ds(base, bpw)], idx_v)
    pltpu.async_copy(op_hbm.at[idx_v], rows_v, sem).wait()
    pltpu.sync_copy(rows_v, out_hbm.at[pl.ds(base, bpw)])


def _combine(out_pad, pos, T):
    _, D = out_pad.shape
    bpw = T // NW
    mesh = plsc.VectorSubcoreMesh(core_axis_name="c", subcore_axis_name="s")
    k = functools.partial(
        pl.kernel,
        mesh=mesh,
        out_type=jax.ShapeDtypeStruct((T, D), jnp.float32),
        scratch_types=[
            pltpu.VMEM((bpw,), jnp.int32),
            pltpu.VMEM((bpw, D), jnp.float32),
            pltpu.SemaphoreType.DMA,
        ],
    )(_combine_body)
    return k(out_pad, pos)


def kernel(inputs, gate_w, W1, W2, W3):
    B, S, D = inputs.shape
    T = B * S
    E = gate_w.shape[0]
    NPAD = T + E * BT
    NT = NPAD // BT
    x = inputs.reshape(T, D)
    pos, te, lv = _routing(x, gate_w, NT)
    x_pad = _dispatch(x, pos, NPAD)
    out_pad = _ffn(te, lv, x_pad, W1, W2, W3)
    out = _combine(out_pad, pos, T)
    return out.reshape(B, S, D)
